# baseline (speedup 1.0000x reference)
import functools

import jax
import jax.numpy as jnp
from jax import lax
from jax.experimental import pallas as pl
from jax.experimental.pallas import tpu as pltpu
from jax.experimental.pallas import tpu_sc as plsc

B = 16384
DIM = 32
VEC_MAX_NORM = 0.1
BIAS_MAX_NORM = 0.1

_NC = 2
_NS = 16
_NW = _NC * _NS
_BPW = B // _NW
_CW = 128
_NCH = _BPW // _CW


def _sc_gather(user_id, movie_id, user_v, movie_v, user_b, movie_b):
    mesh = plsc.VectorSubcoreMesh(core_axis_name="c", subcore_axis_name="s")

    @functools.partial(
        pl.kernel,
        mesh=mesh,
        compiler_params=pltpu.CompilerParams(use_tc_tiling_on_sc=False),
        out_type=[
            jax.ShapeDtypeStruct((B, DIM), jnp.float32),
            jax.ShapeDtypeStruct((B, DIM), jnp.float32),
            jax.ShapeDtypeStruct((B, 1), jnp.float32),
            jax.ShapeDtypeStruct((B, 1), jnp.float32),
        ],
        scratch_types=[
            pltpu.VMEM((_NCH, _CW), jnp.int32),
            pltpu.VMEM((_NCH, _CW), jnp.int32),
            pltpu.VMEM((_BPW, DIM), jnp.float32),
            pltpu.VMEM((_BPW, DIM), jnp.float32),
            pltpu.VMEM((_BPW, 1), jnp.float32),
            pltpu.VMEM((_BPW, 1), jnp.float32),
            pltpu.SemaphoreType.DMA,
            pltpu.SemaphoreType.DMA,
            pltpu.SemaphoreType.DMA,
            pltpu.SemaphoreType.DMA,
        ],
    )
    def gather_k(uid_hbm, mid_hbm, uvt_hbm, mvt_hbm, ubt_hbm, mbt_hbm,
                 uv_out, mv_out, ub_out, mb_out,
                 uidx, midx, uvb, mvb, ubb, mbb, s0, s1, s2, s3):
        wid = lax.axis_index("s") * _NC + lax.axis_index("c")
        base = wid * _BPW
        for j in range(_NCH):
            pltpu.sync_copy(uid_hbm.at[pl.ds(base + j * _CW, _CW)], uidx.at[j])
            pltpu.sync_copy(mid_hbm.at[pl.ds(base + j * _CW, _CW)], midx.at[j])
        copies = []
        for j in range(_NCH):
            copies.append(pltpu.async_copy(
                uvt_hbm.at[uidx.at[j]], uvb.at[pl.ds(j * _CW, _CW)], s0))
            copies.append(pltpu.async_copy(
                mvt_hbm.at[midx.at[j]], mvb.at[pl.ds(j * _CW, _CW)], s1))
            copies.append(pltpu.async_copy(
                ubt_hbm.at[uidx.at[j]], ubb.at[pl.ds(j * _CW, _CW)], s2))
            copies.append(pltpu.async_copy(
                mbt_hbm.at[midx.at[j]], mbb.at[pl.ds(j * _CW, _CW)], s3))
        for c in copies:
            c.wait()
        pltpu.sync_copy(uvb, uv_out.at[pl.ds(base, _BPW)])
        pltpu.sync_copy(mvb, mv_out.at[pl.ds(base, _BPW)])
        pltpu.sync_copy(ubb, ub_out.at[pl.ds(base, _BPW)])
        pltpu.sync_copy(mbb, mb_out.at[pl.ds(base, _BPW)])

    return gather_k(user_id, movie_id, user_v, movie_v, user_b, movie_b)


def _tc_body(uv_ref, mv_ref, ub_ref, mb_ref, b_ref, W1_ref, b1_ref,
             W2_ref, b2_ref, W3_ref, b3_ref, y_ref):
    uv = uv_ref[...]
    mv = mv_ref[...]
    nu = jnp.sqrt(jnp.sum(uv * uv, axis=1, keepdims=True))
    uv = uv * jnp.minimum(1.0, VEC_MAX_NORM / jnp.maximum(nu, 1e-7))
    nm = jnp.sqrt(jnp.sum(mv * mv, axis=1, keepdims=True))
    mv = mv * jnp.minimum(1.0, VEC_MAX_NORM / jnp.maximum(nm, 1e-7))
    ub = ub_ref[...]
    ub = ub * jnp.minimum(1.0, BIAS_MAX_NORM / jnp.maximum(jnp.abs(ub), 1e-7))
    mb = mb_ref[...]
    mb = mb * jnp.minimum(1.0, BIAS_MAX_NORM / jnp.maximum(jnp.abs(mb), 1e-7))
    sum_bias = ub + mb + b_ref[0, 0]
    fm = jnp.sum(uv * mv, axis=1, keepdims=True)
    W1 = W1_ref[...]
    h = jnp.dot(uv, W1[:DIM, :], preferred_element_type=jnp.float32)
    h = h + jnp.dot(mv, W1[DIM:, :], preferred_element_type=jnp.float32)
    h = jnp.maximum(h + b1_ref[...], 0.0)
    h = jnp.maximum(
        jnp.dot(h, W2_ref[...], preferred_element_type=jnp.float32) + b2_ref[...], 0.0)
    deep = jnp.maximum(
        jnp.dot(h, W3_ref[...], preferred_element_type=jnp.float32) + b3_ref[...], 0.0)
    y_ref[...] = jax.nn.sigmoid(sum_bias + fm + deep)


def _tc_fused(uv, mv, ub, mb, b, W1, b1, W2, b2, W3, b3):
    grid = (4,)
    blk = B // grid[0]
    rep = lambda i: (0, 0)
    return pl.pallas_call(
        _tc_body,
        grid=grid,
        in_specs=[
            pl.BlockSpec((blk, DIM), lambda i: (i, 0)),
            pl.BlockSpec((blk, DIM), lambda i: (i, 0)),
            pl.BlockSpec((blk, 1), lambda i: (i, 0)),
            pl.BlockSpec((blk, 1), lambda i: (i, 0)),
            pl.BlockSpec((1, 1), rep),
            pl.BlockSpec((2 * DIM, 16), rep),
            pl.BlockSpec((1, 16), rep),
            pl.BlockSpec((16, 8), rep),
            pl.BlockSpec((1, 8), rep),
            pl.BlockSpec((8, 1), rep),
            pl.BlockSpec((1, 1), rep),
        ],
        out_specs=pl.BlockSpec((blk, 1), lambda i: (i, 0)),
        out_shape=jax.ShapeDtypeStruct((B, 1), jnp.float32),
    )(uv, mv, ub, mb, b, W1, b1, W2, b2, W3, b3)


def kernel(user_id, movie_id, user_v, movie_v, user_b, movie_b, b,
           W1, b1, W2, b2, W3, b3):
    uid = user_id.astype(jnp.int32)
    mid = movie_id.astype(jnp.int32)
    uv, mv, ub, mb = _sc_gather(uid, mid, user_v, movie_v, user_b, movie_b)
    return _tc_fused(
        uv, mv, ub, mb,
        b.reshape(1, 1), W1, b1.reshape(1, 16), W2, b2.reshape(1, 8),
        W3, b3.reshape(1, 1))


# speedup vs baseline: 3.0407x; 3.0407x over previous
import functools

import jax
import jax.numpy as jnp
from jax import lax
from jax.experimental import pallas as pl
from jax.experimental.pallas import tpu as pltpu
from jax.experimental.pallas import tpu_sc as plsc

B = 16384
DIM = 32
VEC_MAX_NORM = 0.1
BIAS_MAX_NORM = 0.1

_NC = 2
_NS = 16
_NW = _NC * _NS
_BPW = B // _NW
_CH = 16
_NP = _BPW // (2 * _CH)


def _sc_gather(user_id, movie_id, uvt, mvt, ub1, mb1):
    mesh = plsc.VectorSubcoreMesh(core_axis_name="c", subcore_axis_name="s")

    @functools.partial(
        pl.kernel,
        mesh=mesh,
        compiler_params=pltpu.CompilerParams(use_tc_tiling_on_sc=True),
        out_type=[
            jax.ShapeDtypeStruct((B, DIM), jnp.float32),
            jax.ShapeDtypeStruct((B, DIM), jnp.float32),
            jax.ShapeDtypeStruct((B,), jnp.float32),
            jax.ShapeDtypeStruct((B,), jnp.float32),
        ],
        scratch_types=[
            pltpu.VMEM((_BPW,), jnp.int32),
            pltpu.VMEM((_BPW,), jnp.int32),
            pltpu.VMEM((2 * _CH * 8, DIM), jnp.float32),
            pltpu.VMEM((2 * _CH * 8, DIM), jnp.float32),
            pltpu.VMEM((2 * _CH, DIM), jnp.float32),
            pltpu.VMEM((2 * _CH, DIM), jnp.float32),
            pltpu.VMEM((_BPW,), jnp.float32),
            pltpu.VMEM((_BPW,), jnp.float32),
            pltpu.SemaphoreType.DMA,
            pltpu.SemaphoreType.DMA,
            pltpu.SemaphoreType.DMA,
            pltpu.SemaphoreType.DMA,
            pltpu.SemaphoreType.DMA,
        ],
    )
    def gather_k(uid_hbm, mid_hbm, uvt_hbm, mvt_hbm, ubt_hbm, mbt_hbm,
                 uv_out, mv_out, ub_out, mb_out,
                 uids, mids, ubuf, mbuf, urows, mrows, ubb, mbb,
                 sua, sub, sma, smb, sbias):
        wid = lax.axis_index("s") * _NC + lax.axis_index("c")
        base = wid * _BPW
        pltpu.sync_copy(uid_hbm.at[pl.ds(base, _BPW)], uids)
        pltpu.sync_copy(mid_hbm.at[pl.ds(base, _BPW)], mids)

        bias_copies = []
        for j in range(_BPW // 128):
            bias_copies.append(pltpu.async_copy(
                ubt_hbm.at[uids.at[pl.ds(j * 128, 128)]],
                ubb.at[pl.ds(j * 128, 128)], sbias))
            bias_copies.append(pltpu.async_copy(
                mbt_hbm.at[mids.at[pl.ds(j * 128, 128)]],
                mbb.at[pl.ds(j * 128, 128)], sbias))

        def issue(ids_v, t_hbm, gbuf, coff, buf, sem):
            for g in range(_CH // 16):
                r = ids_v[pl.ds(coff + g * 16, 16)]
                gvec = (r >> 3) * 8
                for l in range(16):
                    off = pl.multiple_of(gvec[l], 8)
                    d = buf * (_CH * 8) + (g * 16 + l) * 8
                    pltpu.async_copy(
                        t_hbm.at[pl.ds(off, 8)], gbuf.at[pl.ds(d, 8)], sem)

        def drain(t_hbm, gbuf, sem):
            pltpu.make_async_copy(
                t_hbm.at[pl.ds(0, _CH * 8)],
                gbuf.at[pl.ds(0, _CH * 8)], sem).wait()

        def extract(ids_v, gbuf, rows, coff, buf):
            for g in range(_CH // 16):
                r = ids_v[pl.ds(coff + g * 16, 16)]
                svec = (r & 7) + (g * 16 + lax.iota(jnp.int32, 16)) * 8
                for l in range(16):
                    s = buf * (_CH * 8) + svec[l]
                    d = buf * _CH + g * 16 + l
                    for h in range(2):
                        rows[d, pl.ds(h * 16, 16)] = gbuf[s, pl.ds(h * 16, 16)]

        def pair(t, _):
            c0 = pl.multiple_of(t * 2 * _CH, 2 * _CH)
            c1 = c0 + _CH
            issue(uids, uvt_hbm, ubuf, c0, 0, sua)
            issue(mids, mvt_hbm, mbuf, c0, 0, sma)
            issue(uids, uvt_hbm, ubuf, c1, 1, sub)
            issue(mids, mvt_hbm, mbuf, c1, 1, smb)
            drain(uvt_hbm, ubuf, sua)
            extract(uids, ubuf, urows, c0, 0)
            drain(mvt_hbm, mbuf, sma)
            extract(mids, mbuf, mrows, c0, 0)
            drain(uvt_hbm, ubuf, sub)
            extract(uids, ubuf, urows, c1, 1)
            drain(mvt_hbm, mbuf, smb)
            extract(mids, mbuf, mrows, c1, 1)
            pltpu.sync_copy(urows, uv_out.at[pl.ds(base + c0, 2 * _CH)])
            pltpu.sync_copy(mrows, mv_out.at[pl.ds(base + c0, 2 * _CH)])
            return _

        lax.fori_loop(0, _NP, pair, None)

        for cp in bias_copies:
            cp.wait()
        pltpu.sync_copy(ubb, ub_out.at[pl.ds(base, _BPW)])
        pltpu.sync_copy(mbb, mb_out.at[pl.ds(base, _BPW)])

    return gather_k(user_id, movie_id, uvt, mvt, ub1, mb1)


def _tc_body(uv_ref, mv_ref, ub_ref, mb_ref, b_ref,
             W1_ref, b1_ref, W2_ref, b2_ref, W3_ref, b3_ref, y_ref):
    uv = uv_ref[...]
    mv = mv_ref[...]
    nu = jnp.sqrt(jnp.sum(uv * uv, axis=1, keepdims=True))
    uv = uv * jnp.minimum(1.0, VEC_MAX_NORM / jnp.maximum(nu, 1e-7))
    nm = jnp.sqrt(jnp.sum(mv * mv, axis=1, keepdims=True))
    mv = mv * jnp.minimum(1.0, VEC_MAX_NORM / jnp.maximum(nm, 1e-7))
    ub = ub_ref[...]
    ub = ub * jnp.minimum(1.0, BIAS_MAX_NORM / jnp.maximum(jnp.abs(ub), 1e-7))
    mb = mb_ref[...]
    mb = mb * jnp.minimum(1.0, BIAS_MAX_NORM / jnp.maximum(jnp.abs(mb), 1e-7))
    sum_bias = ub + mb + b_ref[0, 0]
    fm = jnp.sum(uv * mv, axis=1, keepdims=True)
    W1 = W1_ref[...]
    h = jnp.dot(uv, W1[:DIM, :], preferred_element_type=jnp.float32)
    h = h + jnp.dot(mv, W1[DIM:, :], preferred_element_type=jnp.float32)
    h = jnp.maximum(h + b1_ref[...], 0.0)
    h = jnp.maximum(
        jnp.dot(h, W2_ref[...], preferred_element_type=jnp.float32) + b2_ref[...], 0.0)
    deep = jnp.maximum(
        jnp.dot(h, W3_ref[...], preferred_element_type=jnp.float32) + b3_ref[...], 0.0)
    y_ref[...] = jax.nn.sigmoid(sum_bias + fm + deep)


def _tc_fused(uv, mv, ub, mb, b, W1, b1, W2, b2, W3, b3):
    grid = (4,)
    blk = B // grid[0]
    rep = lambda i: (0, 0)
    return pl.pallas_call(
        _tc_body,
        grid=grid,
        in_specs=[
            pl.BlockSpec((blk, DIM), lambda i: (i, 0)),
            pl.BlockSpec((blk, DIM), lambda i: (i, 0)),
            pl.BlockSpec((blk, 1), lambda i: (i, 0)),
            pl.BlockSpec((blk, 1), lambda i: (i, 0)),
            pl.BlockSpec((1, 1), rep),
            pl.BlockSpec((2 * DIM, 16), rep),
            pl.BlockSpec((1, 16), rep),
            pl.BlockSpec((16, 8), rep),
            pl.BlockSpec((1, 8), rep),
            pl.BlockSpec((8, 1), rep),
            pl.BlockSpec((1, 1), rep),
        ],
        out_specs=pl.BlockSpec((blk, 1), lambda i: (i, 0)),
        out_shape=jax.ShapeDtypeStruct((B, 1), jnp.float32),
    )(uv, mv, ub, mb, b, W1, b1, W2, b2, W3, b3)


def kernel(user_id, movie_id, user_v, movie_v, user_b, movie_b, b,
           W1, b1, W2, b2, W3, b3):
    uid = user_id.astype(jnp.int32)
    mid = movie_id.astype(jnp.int32)
    uv, mv, ub, mb = _sc_gather(
        uid, mid, user_v, movie_v,
        user_b.reshape(-1), movie_b.reshape(-1))
    return _tc_fused(
        uv, mv, ub.reshape(B, 1), mb.reshape(B, 1),
        b.reshape(1, 1), W1, b1.reshape(1, 16), W2, b2.reshape(1, 8),
        W3, b3.reshape(1, 1))


# speedup vs baseline: 3.1915x; 1.0496x over previous
import functools

import jax
import jax.numpy as jnp
from jax import lax
from jax.experimental import pallas as pl
from jax.experimental.pallas import tpu as pltpu
from jax.experimental.pallas import tpu_sc as plsc

B = 16384
DIM = 32
VEC_MAX_NORM = 0.1
BIAS_MAX_NORM = 0.1

_NC = 2
_NS = 16
_NW = _NC * _NS
_BPW = B // _NW
_CH = 32
_NCH = _BPW // _CH


def _sc_gather(ids, vt, b1):
    mesh = plsc.VectorSubcoreMesh(core_axis_name="c", subcore_axis_name="s")

    @functools.partial(
        pl.kernel,
        mesh=mesh,
        compiler_params=pltpu.CompilerParams(use_tc_tiling_on_sc=True),
        out_type=[
            jax.ShapeDtypeStruct((B, DIM), jnp.float32),
            jax.ShapeDtypeStruct((B,), jnp.float32),
        ],
        scratch_types=[
            pltpu.VMEM((_BPW,), jnp.int32),
            pltpu.VMEM((2 * _CH * 8, DIM), jnp.float32),
            pltpu.VMEM((_CH, DIM), jnp.float32),
            pltpu.VMEM((_BPW,), jnp.float32),
            pltpu.SemaphoreType.DMA,
            pltpu.SemaphoreType.DMA,
            pltpu.SemaphoreType.DMA,
        ],
    )
    def gather_k(ids_hbm, vt_hbm, bt_hbm, v_out, b_out,
                 idsv, gbuf, rows, bb, sa, sb, sbias):
        wid = lax.axis_index("s") * _NC + lax.axis_index("c")
        base = wid * _BPW
        pltpu.sync_copy(ids_hbm.at[pl.ds(base, _BPW)], idsv)

        bias_copies = []
        for j in range(_BPW // 128):
            bias_copies.append(pltpu.async_copy(
                bt_hbm.at[idsv.at[pl.ds(j * 128, 128)]],
                bb.at[pl.ds(j * 128, 128)], sbias))

        def issue(coff, buf, sem):
            for g in range(_CH // 16):
                r = idsv[pl.ds(coff + g * 16, 16)]
                gvec = (r >> 3) * 8
                for l in range(16):
                    off = pl.multiple_of(gvec[l], 8)
                    d = buf * (_CH * 8) + (g * 16 + l) * 8
                    pltpu.async_copy(
                        vt_hbm.at[pl.ds(off, 8)], gbuf.at[pl.ds(d, 8)], sem)

        def drain_extract(coff, buf, sem):
            pltpu.make_async_copy(
                vt_hbm.at[pl.ds(0, _CH * 8)],
                gbuf.at[pl.ds(buf * (_CH * 8), _CH * 8)], sem).wait()
            for g in range(_CH // 16):
                r = idsv[pl.ds(coff + g * 16, 16)]
                svec = (r & 7) + (g * 16 + lax.iota(jnp.int32, 16)) * 8
                for l in range(16):
                    s = buf * (_CH * 8) + svec[l]
                    d = g * 16 + l
                    for h in range(2):
                        rows[d, pl.ds(h * 16, 16)] = gbuf[s, pl.ds(h * 16, 16)]

        def body(t, _):
            @pl.when(t < _NCH)
            def _():
                coff = pl.multiple_of(t * _CH, _CH)
                @pl.when(t % 2 == 0)
                def _():
                    issue(coff, 0, sa)
                @pl.when(t % 2 == 1)
                def _():
                    issue(coff, 1, sb)
            @pl.when(t > 0)
            def _():
                tp = t - 1
                coffp = pl.multiple_of(tp * _CH, _CH)
                @pl.when(tp % 2 == 0)
                def _():
                    drain_extract(coffp, 0, sa)
                @pl.when(tp % 2 == 1)
                def _():
                    drain_extract(coffp, 1, sb)
                pltpu.sync_copy(rows, v_out.at[pl.ds(base + coffp, _CH)])
            return _

        lax.fori_loop(0, _NCH + 1, body, None)

        for cp in bias_copies:
            cp.wait()
        pltpu.sync_copy(bb, b_out.at[pl.ds(base, _BPW)])

    return gather_k(ids, vt, b1)


def _tc_body(uv_ref, mv_ref, ub_ref, mb_ref, b_ref,
             W1_ref, b1_ref, W2_ref, b2_ref, W3_ref, b3_ref, y_ref):
    uv = uv_ref[...]
    mv = mv_ref[...]
    nu = jnp.sqrt(jnp.sum(uv * uv, axis=1, keepdims=True))
    uv = uv * jnp.minimum(1.0, VEC_MAX_NORM / jnp.maximum(nu, 1e-7))
    nm = jnp.sqrt(jnp.sum(mv * mv, axis=1, keepdims=True))
    mv = mv * jnp.minimum(1.0, VEC_MAX_NORM / jnp.maximum(nm, 1e-7))
    ub = ub_ref[...]
    ub = ub * jnp.minimum(1.0, BIAS_MAX_NORM / jnp.maximum(jnp.abs(ub), 1e-7))
    mb = mb_ref[...]
    mb = mb * jnp.minimum(1.0, BIAS_MAX_NORM / jnp.maximum(jnp.abs(mb), 1e-7))
    sum_bias = ub + mb + b_ref[0, 0]
    fm = jnp.sum(uv * mv, axis=1, keepdims=True)
    W1 = W1_ref[...]
    h = jnp.dot(uv, W1[:DIM, :], preferred_element_type=jnp.float32)
    h = h + jnp.dot(mv, W1[DIM:, :], preferred_element_type=jnp.float32)
    h = jnp.maximum(h + b1_ref[...], 0.0)
    h = jnp.maximum(
        jnp.dot(h, W2_ref[...], preferred_element_type=jnp.float32) + b2_ref[...], 0.0)
    deep = jnp.maximum(
        jnp.dot(h, W3_ref[...], preferred_element_type=jnp.float32) + b3_ref[...], 0.0)
    y_ref[...] = jax.nn.sigmoid(sum_bias + fm + deep)


def _tc_fused(uv, mv, ub, mb, b, W1, b1, W2, b2, W3, b3):
    grid = (4,)
    blk = B // grid[0]
    rep = lambda i: (0, 0)
    return pl.pallas_call(
        _tc_body,
        grid=grid,
        in_specs=[
            pl.BlockSpec((blk, DIM), lambda i: (i, 0)),
            pl.BlockSpec((blk, DIM), lambda i: (i, 0)),
            pl.BlockSpec((blk, 1), lambda i: (i, 0)),
            pl.BlockSpec((blk, 1), lambda i: (i, 0)),
            pl.BlockSpec((1, 1), rep),
            pl.BlockSpec((2 * DIM, 16), rep),
            pl.BlockSpec((1, 16), rep),
            pl.BlockSpec((16, 8), rep),
            pl.BlockSpec((1, 8), rep),
            pl.BlockSpec((8, 1), rep),
            pl.BlockSpec((1, 1), rep),
        ],
        out_specs=pl.BlockSpec((blk, 1), lambda i: (i, 0)),
        out_shape=jax.ShapeDtypeStruct((B, 1), jnp.float32),
    )(uv, mv, ub, mb, b, W1, b1, W2, b2, W3, b3)


def kernel(user_id, movie_id, user_v, movie_v, user_b, movie_b, b,
           W1, b1, W2, b2, W3, b3):
    uid = user_id.astype(jnp.int32)
    mid = movie_id.astype(jnp.int32)
    uv, ub = _sc_gather(uid, user_v, user_b.reshape(-1))
    mv, mb = _sc_gather(mid, movie_v, movie_b.reshape(-1))
    return _tc_fused(
        uv, mv, ub.reshape(B, 1), mb.reshape(B, 1),
        b.reshape(1, 1), W1, b1.reshape(1, 16), W2, b2.reshape(1, 8),
        W3, b3.reshape(1, 1))


# speedup vs baseline: 3.4870x; 1.0926x over previous
import functools

import jax
import jax.numpy as jnp
from jax import lax
from jax.experimental import pallas as pl
from jax.experimental.pallas import tpu as pltpu
from jax.experimental.pallas import tpu_sc as plsc

B = 16384
DIM = 32
VEC_MAX_NORM = 0.1
BIAS_MAX_NORM = 0.1

_NC = 2
_NS = 16
_NW = _NC * _NS
_BPW = B // _NW
_CH = 32
_NCH = _BPW // _CH


def _sc_gather(ids, vt, b1):
    mesh = plsc.VectorSubcoreMesh(core_axis_name="c", subcore_axis_name="s")

    @functools.partial(
        pl.kernel,
        mesh=mesh,
        compiler_params=pltpu.CompilerParams(use_tc_tiling_on_sc=True),
        out_type=[
            jax.ShapeDtypeStruct((B, DIM), jnp.float32),
            jax.ShapeDtypeStruct((B,), jnp.float32),
        ],
        scratch_types=[
            pltpu.VMEM((_BPW,), jnp.int32),
            pltpu.VMEM((2 * _CH, DIM), jnp.float32),
            pltpu.VMEM((_BPW,), jnp.float32),
            pltpu.SemaphoreType.DMA,
            pltpu.SemaphoreType.DMA,
            pltpu.SemaphoreType.DMA,
        ],
    )
    def gather_k(ids_hbm, vt_hbm, bt_hbm, v_out, b_out,
                 idsv, gbuf, bb, sa, sb, sbias):
        wid = lax.axis_index("s") * _NC + lax.axis_index("c")
        base = wid * _BPW
        pltpu.sync_copy(ids_hbm.at[pl.ds(base, _BPW)], idsv)

        bias_copies = []
        for j in range(_BPW // 128):
            bias_copies.append(pltpu.async_copy(
                bt_hbm.at[idsv.at[pl.ds(j * 128, 128)]],
                bb.at[pl.ds(j * 128, 128)], sbias))

        def issue(coff, buf, sem):
            for g in range(_CH // 16):
                r = idsv[pl.ds(coff + g * 16, 16)]
                for l in range(16):
                    off = r[l]
                    d = buf * _CH + g * 16 + l
                    pltpu.async_copy(
                        vt_hbm.at[pl.ds(off, 1)],
                        gbuf.at[pl.ds(d, 1)], sem)

        def drain(buf, sem):
            pltpu.make_async_copy(
                vt_hbm.at[pl.ds(0, _CH)],
                gbuf.at[pl.ds(buf * _CH, _CH)], sem).wait()

        def body(t, _):
            @pl.when(t < _NCH)
            def _():
                coff = pl.multiple_of(t * _CH, _CH)
                @pl.when(t % 2 == 0)
                def _():
                    issue(coff, 0, sa)
                @pl.when(t % 2 == 1)
                def _():
                    issue(coff, 1, sb)
            @pl.when(t > 0)
            def _():
                tp = t - 1
                coffp = pl.multiple_of(tp * _CH, _CH)
                @pl.when(tp % 2 == 0)
                def _():
                    drain(0, sa)
                    pltpu.sync_copy(
                        gbuf.at[pl.ds(0, _CH)],
                        v_out.at[pl.ds(base + coffp, _CH)])
                @pl.when(tp % 2 == 1)
                def _():
                    drain(1, sb)
                    pltpu.sync_copy(
                        gbuf.at[pl.ds(_CH, _CH)],
                        v_out.at[pl.ds(base + coffp, _CH)])
            return _

        lax.fori_loop(0, _NCH + 1, body, None)

        for cp in bias_copies:
            cp.wait()
        pltpu.sync_copy(bb, b_out.at[pl.ds(base, _BPW)])

    return gather_k(ids, vt, b1)


def _tc_body(uv_ref, mv_ref, ub_ref, mb_ref, b_ref,
             W1_ref, b1_ref, W2_ref, b2_ref, W3_ref, b3_ref, y_ref):
    uv = uv_ref[...]
    mv = mv_ref[...]
    nu = jnp.sqrt(jnp.sum(uv * uv, axis=1, keepdims=True))
    uv = uv * jnp.minimum(1.0, VEC_MAX_NORM / jnp.maximum(nu, 1e-7))
    nm = jnp.sqrt(jnp.sum(mv * mv, axis=1, keepdims=True))
    mv = mv * jnp.minimum(1.0, VEC_MAX_NORM / jnp.maximum(nm, 1e-7))
    ub = ub_ref[...]
    ub = ub * jnp.minimum(1.0, BIAS_MAX_NORM / jnp.maximum(jnp.abs(ub), 1e-7))
    mb = mb_ref[...]
    mb = mb * jnp.minimum(1.0, BIAS_MAX_NORM / jnp.maximum(jnp.abs(mb), 1e-7))
    sum_bias = ub + mb + b_ref[0, 0]
    fm = jnp.sum(uv * mv, axis=1, keepdims=True)
    W1 = W1_ref[...]
    h = jnp.dot(uv, W1[:DIM, :], preferred_element_type=jnp.float32)
    h = h + jnp.dot(mv, W1[DIM:, :], preferred_element_type=jnp.float32)
    h = jnp.maximum(h + b1_ref[...], 0.0)
    h = jnp.maximum(
        jnp.dot(h, W2_ref[...], preferred_element_type=jnp.float32) + b2_ref[...], 0.0)
    deep = jnp.maximum(
        jnp.dot(h, W3_ref[...], preferred_element_type=jnp.float32) + b3_ref[...], 0.0)
    y_ref[...] = jax.nn.sigmoid(sum_bias + fm + deep)


def _tc_fused(uv, mv, ub, mb, b, W1, b1, W2, b2, W3, b3):
    grid = (4,)
    blk = B // grid[0]
    rep = lambda i: (0, 0)
    return pl.pallas_call(
        _tc_body,
        grid=grid,
        in_specs=[
            pl.BlockSpec((blk, DIM), lambda i: (i, 0)),
            pl.BlockSpec((blk, DIM), lambda i: (i, 0)),
            pl.BlockSpec((blk, 1), lambda i: (i, 0)),
            pl.BlockSpec((blk, 1), lambda i: (i, 0)),
            pl.BlockSpec((1, 1), rep),
            pl.BlockSpec((2 * DIM, 16), rep),
            pl.BlockSpec((1, 16), rep),
            pl.BlockSpec((16, 8), rep),
            pl.BlockSpec((1, 8), rep),
            pl.BlockSpec((8, 1), rep),
            pl.BlockSpec((1, 1), rep),
        ],
        out_specs=pl.BlockSpec((blk, 1), lambda i: (i, 0)),
        out_shape=jax.ShapeDtypeStruct((B, 1), jnp.float32),
    )(uv, mv, ub, mb, b, W1, b1, W2, b2, W3, b3)


def kernel(user_id, movie_id, user_v, movie_v, user_b, movie_b, b,
           W1, b1, W2, b2, W3, b3):
    uid = user_id.astype(jnp.int32)
    mid = movie_id.astype(jnp.int32)
    uv, ub = _sc_gather(uid, user_v, user_b.reshape(-1))
    mv, mb = _sc_gather(mid, movie_v, movie_b.reshape(-1))
    return _tc_fused(
        uv, mv, ub.reshape(B, 1), mb.reshape(B, 1),
        b.reshape(1, 1), W1, b1.reshape(1, 16), W2, b2.reshape(1, 8),
        W3, b3.reshape(1, 1))


# speedup vs baseline: 3.4953x; 1.0024x over previous
import functools

import jax
import jax.numpy as jnp
from jax import lax
from jax.experimental import pallas as pl
from jax.experimental.pallas import tpu as pltpu
from jax.experimental.pallas import tpu_sc as plsc

B = 16384
DIM = 32
VEC_MAX_NORM = 0.1
BIAS_MAX_NORM = 0.1

_NC = 2
_NS = 16
_NW = _NC * _NS
_BPW = B // _NW
_CH = 64
_NCH = _BPW // _CH


def _sc_gather(ids, vt, b1):
    mesh = plsc.VectorSubcoreMesh(core_axis_name="c", subcore_axis_name="s")

    @functools.partial(
        pl.kernel,
        mesh=mesh,
        compiler_params=pltpu.CompilerParams(use_tc_tiling_on_sc=True),
        out_type=[
            jax.ShapeDtypeStruct((B, DIM), jnp.float32),
            jax.ShapeDtypeStruct((B,), jnp.float32),
        ],
        scratch_types=[
            pltpu.VMEM((_BPW,), jnp.int32),
            pltpu.VMEM((2 * _CH, DIM), jnp.float32),
            pltpu.VMEM((_BPW,), jnp.float32),
            pltpu.SemaphoreType.DMA,
            pltpu.SemaphoreType.DMA,
            pltpu.SemaphoreType.DMA,
        ],
    )
    def gather_k(ids_hbm, vt_hbm, bt_hbm, v_out, b_out,
                 idsv, gbuf, bb, sa, sb, sbias):
        wid = lax.axis_index("s") * _NC + lax.axis_index("c")
        base = wid * _BPW
        pltpu.sync_copy(ids_hbm.at[pl.ds(base, _BPW)], idsv)

        bias_copies = []
        for j in range(_BPW // 128):
            bias_copies.append(pltpu.async_copy(
                bt_hbm.at[idsv.at[pl.ds(j * 128, 128)]],
                bb.at[pl.ds(j * 128, 128)], sbias))

        def issue(coff, buf, sem):
            for g in range(_CH // 16):
                r = idsv[pl.ds(coff + g * 16, 16)]
                for l in range(16):
                    off = r[l]
                    d = buf * _CH + g * 16 + l
                    pltpu.async_copy(
                        vt_hbm.at[pl.ds(off, 1)],
                        gbuf.at[pl.ds(d, 1)], sem)

        def drain(buf, sem):
            pltpu.make_async_copy(
                vt_hbm.at[pl.ds(0, _CH)],
                gbuf.at[pl.ds(buf * _CH, _CH)], sem).wait()

        def body(t, _):
            @pl.when(t < _NCH)
            def _():
                coff = pl.multiple_of(t * _CH, _CH)
                @pl.when(t % 2 == 0)
                def _():
                    issue(coff, 0, sa)
                @pl.when(t % 2 == 1)
                def _():
                    issue(coff, 1, sb)
            @pl.when(t > 0)
            def _():
                tp = t - 1
                coffp = pl.multiple_of(tp * _CH, _CH)
                @pl.when(tp % 2 == 0)
                def _():
                    drain(0, sa)
                    pltpu.sync_copy(
                        gbuf.at[pl.ds(0, _CH)],
                        v_out.at[pl.ds(base + coffp, _CH)])
                @pl.when(tp % 2 == 1)
                def _():
                    drain(1, sb)
                    pltpu.sync_copy(
                        gbuf.at[pl.ds(_CH, _CH)],
                        v_out.at[pl.ds(base + coffp, _CH)])
            return _

        lax.fori_loop(0, _NCH + 1, body, None)

        for cp in bias_copies:
            cp.wait()
        pltpu.sync_copy(bb, b_out.at[pl.ds(base, _BPW)])

    return gather_k(ids, vt, b1)


def _tc_body(uv_ref, mv_ref, ub_ref, mb_ref, b_ref,
             W1_ref, b1_ref, W2_ref, b2_ref, W3_ref, b3_ref, y_ref):
    uv = uv_ref[...]
    mv = mv_ref[...]
    nu = jnp.sqrt(jnp.sum(uv * uv, axis=1, keepdims=True))
    uv = uv * jnp.minimum(1.0, VEC_MAX_NORM / jnp.maximum(nu, 1e-7))
    nm = jnp.sqrt(jnp.sum(mv * mv, axis=1, keepdims=True))
    mv = mv * jnp.minimum(1.0, VEC_MAX_NORM / jnp.maximum(nm, 1e-7))
    ub = ub_ref[...]
    ub = ub * jnp.minimum(1.0, BIAS_MAX_NORM / jnp.maximum(jnp.abs(ub), 1e-7))
    mb = mb_ref[...]
    mb = mb * jnp.minimum(1.0, BIAS_MAX_NORM / jnp.maximum(jnp.abs(mb), 1e-7))
    sum_bias = ub + mb + b_ref[0, 0]
    fm = jnp.sum(uv * mv, axis=1, keepdims=True)
    W1 = W1_ref[...]
    h = jnp.dot(uv, W1[:DIM, :], preferred_element_type=jnp.float32)
    h = h + jnp.dot(mv, W1[DIM:, :], preferred_element_type=jnp.float32)
    h = jnp.maximum(h + b1_ref[...], 0.0)
    h = jnp.maximum(
        jnp.dot(h, W2_ref[...], preferred_element_type=jnp.float32) + b2_ref[...], 0.0)
    deep = jnp.maximum(
        jnp.dot(h, W3_ref[...], preferred_element_type=jnp.float32) + b3_ref[...], 0.0)
    y_ref[...] = jax.nn.sigmoid(sum_bias + fm + deep)


def _tc_fused(uv, mv, ub, mb, b, W1, b1, W2, b2, W3, b3):
    grid = (4,)
    blk = B // grid[0]
    rep = lambda i: (0, 0)
    return pl.pallas_call(
        _tc_body,
        grid=grid,
        in_specs=[
            pl.BlockSpec((blk, DIM), lambda i: (i, 0)),
            pl.BlockSpec((blk, DIM), lambda i: (i, 0)),
            pl.BlockSpec((blk, 1), lambda i: (i, 0)),
            pl.BlockSpec((blk, 1), lambda i: (i, 0)),
            pl.BlockSpec((1, 1), rep),
            pl.BlockSpec((2 * DIM, 16), rep),
            pl.BlockSpec((1, 16), rep),
            pl.BlockSpec((16, 8), rep),
            pl.BlockSpec((1, 8), rep),
            pl.BlockSpec((8, 1), rep),
            pl.BlockSpec((1, 1), rep),
        ],
        out_specs=pl.BlockSpec((blk, 1), lambda i: (i, 0)),
        out_shape=jax.ShapeDtypeStruct((B, 1), jnp.float32),
    )(uv, mv, ub, mb, b, W1, b1, W2, b2, W3, b3)


def kernel(user_id, movie_id, user_v, movie_v, user_b, movie_b, b,
           W1, b1, W2, b2, W3, b3):
    uid = user_id.astype(jnp.int32)
    mid = movie_id.astype(jnp.int32)
    uv, ub = _sc_gather(uid, user_v, user_b.reshape(-1))
    mv, mb = _sc_gather(mid, movie_v, movie_b.reshape(-1))
    return _tc_fused(
        uv, mv, ub.reshape(B, 1), mb.reshape(B, 1),
        b.reshape(1, 1), W1, b1.reshape(1, 16), W2, b2.reshape(1, 8),
        W3, b3.reshape(1, 1))
